# Initial kernel scaffold; baseline (speedup 1.0000x reference)
import jax
import jax.numpy as jnp
from jax.experimental import pallas as pl


def kernel(x, qW, qb, kW, kb, phi1W, phi1b, phi2W, phi2b, xi1W, xi1b, xi2W, xi2b, rho1W, rho1b, rho2W, rho2b):
    raise NotImplementedError("write your pallas kernel here")


# speedup vs baseline: 1.0002x; 1.0002x over previous
import functools

import jax
import jax.numpy as jnp
from jax.experimental import pallas as pl
from jax.experimental.pallas import tpu as pltpu

B, L, D = 8, 2048, 128
H = 128
TOP_K = 512
SCALE = H ** -0.5


def _scores_body(x_ref, qW_ref, qb_ref, kW_ref, kb_ref, s_ref):
    xb = x_ref[0]
    q = jax.lax.dot_general(xb, qW_ref[...], (((1,), (1,)), ((), ())),
                            preferred_element_type=jnp.float32) + qb_ref[...]
    k = jax.lax.dot_general(xb, kW_ref[...], (((1,), (1,)), ((), ())),
                            preferred_element_type=jnp.float32) + kb_ref[...]
    s = jax.lax.dot_general(q, k, (((1,), (1,)), ((), ())),
                            preferred_element_type=jnp.float32) * SCALE
    valid = jnp.sum(jnp.abs(xb), axis=1) != 0.0
    m2 = valid[:, None] & valid[None, :]
    s_ref[0] = jnp.where(m2, s, -jnp.inf)


def _scores(x, qW, qb, kW, kb):
    return pl.pallas_call(
        _scores_body,
        grid=(B,),
        in_specs=[
            pl.BlockSpec((1, L, D), lambda b: (b, 0, 0)),
            pl.BlockSpec((H, D), lambda b: (0, 0)),
            pl.BlockSpec((1, H), lambda b: (0, 0)),
            pl.BlockSpec((H, D), lambda b: (0, 0)),
            pl.BlockSpec((1, H), lambda b: (0, 0)),
        ],
        out_specs=pl.BlockSpec((1, L, L), lambda b: (b, 0, 0)),
        out_shape=jax.ShapeDtypeStruct((B, L, L), jnp.float32),
    )(x, qW, qb.reshape(1, H), kW, kb.reshape(1, H))


def _mlp2(h, W1, b1, W2, b2):
    h = jax.nn.relu(h @ W1.T + b1)
    return h @ W2.T + b2


def kernel(x, qW, qb, kW, kb, phi1W, phi1b, phi2W, phi2b,
           xi1W, xi1b, xi2W, xi2b, rho1W, rho1b, rho2W, rho2b):
    scores = _scores(x, qW, qb, kW, kb)
    flat = scores.reshape(B, -1)
    topk_vals, topk_idx = jax.lax.top_k(flat, TOP_K)
    weights = jax.nn.softmax(topk_vals, axis=-1)
    row = topk_idx // L
    col = topk_idx % L
    self_mask = row == col
    x_i = jnp.take_along_axis(x, row[:, :, None], axis=1)
    x_j = jnp.take_along_axis(x, col[:, :, None], axis=1)
    feat_self = _mlp2(x_i, phi1W, phi1b, phi2W, phi2b)
    feat_pair = _mlp2(jnp.concatenate([x_i, x_j], axis=-1), xi1W, xi1b, xi2W, xi2b)
    interaction = jnp.where(self_mask[:, :, None], feat_self, feat_pair)
    pooled = jnp.sum(interaction * weights[:, :, None], axis=1)
    return _mlp2(pooled, rho1W, rho1b, rho2W, rho2b)


# speedup vs baseline: 76.5364x; 76.5219x over previous
import numpy as np

import jax
import jax.numpy as jnp
from jax import lax
from jax.experimental import pallas as pl
from jax.experimental.pallas import tpu as pltpu
from jax.experimental.pallas import tpu_sc as plsc

B, L, D = 8, 2048, 128
H = 128
TOP_K = 512
SCALE = H ** -0.5
LL = L * L
NG = L * 128
GSH = NG // 16
GCAP = 512
CCAP = 1024
I32_MIN = -2147483648
M31 = 0x7FFFFFFF


def _scores_body(x_ref, qW_ref, qb_ref, kW_ref, kb_ref, s_ref, g_ref):
    xb = x_ref[0]
    q = lax.dot_general(xb, qW_ref[...], (((1,), (1,)), ((), ())),
                        preferred_element_type=jnp.float32) + qb_ref[...]
    k = lax.dot_general(xb, kW_ref[...], (((1,), (1,)), ((), ())),
                        preferred_element_type=jnp.float32) + kb_ref[...]
    s = lax.dot_general(q, k, (((1,), (1,)), ((), ())),
                        preferred_element_type=jnp.float32) * SCALE
    valid = jnp.sum(jnp.abs(xb), axis=1) != 0.0
    s = jnp.where(valid[:, None] & valid[None, :], s, -jnp.inf)
    s_ref[0] = s
    g = s[:, 0:128]
    for l in range(1, 16):
        g = jnp.maximum(g, s[:, 128 * l:128 * (l + 1)])
    g_ref[0] = g


def _scores(x, qW, qb, kW, kb):
    return pl.pallas_call(
        _scores_body,
        grid=(B,),
        in_specs=[
            pl.BlockSpec((1, L, D), lambda b: (b, 0, 0)),
            pl.BlockSpec((H, D), lambda b: (0, 0)),
            pl.BlockSpec((1, H), lambda b: (0, 0)),
            pl.BlockSpec((H, D), lambda b: (0, 0)),
            pl.BlockSpec((1, H), lambda b: (0, 0)),
        ],
        out_specs=[
            pl.BlockSpec((1, L, L), lambda b: (b, 0, 0)),
            pl.BlockSpec((1, L, 128), lambda b: (b, 0, 0)),
        ],
        out_shape=[
            jax.ShapeDtypeStruct((B, L, L), jnp.float32),
            jax.ShapeDtypeStruct((B, L, 128), jnp.float32),
        ],
    )(x, qW, qb.reshape(1, H), kW, kb.reshape(1, H))


def _theta_body(g_ref, th_ref):
    g = g_ref[0]
    bb = lax.bitcast_convert_type(g, jnp.int32)
    keys = bb ^ (lax.shift_right_arithmetic(bb, 31) & M31)

    def bit_step(i, toff):
        bit = lax.shift_left(jnp.int32(1), 31 - i)
        cand = toff | bit
        thr = cand ^ I32_MIN
        cnt = jnp.sum((keys >= thr).astype(jnp.int32))
        return lax.select(cnt >= TOP_K, cand, toff)

    toff = lax.fori_loop(0, 32, bit_step, np.int32(0))
    vk = toff ^ I32_MIN
    fb = jnp.where(vk < 0, vk ^ M31, vk)
    th = lax.bitcast_convert_type(fb, jnp.float32)
    th_ref[0] = jnp.full((8, 128), th, jnp.float32)


def _theta(gmax):
    return pl.pallas_call(
        _theta_body,
        grid=(B,),
        in_specs=[pl.BlockSpec((1, L, 128), lambda b: (b, 0, 0))],
        out_specs=pl.BlockSpec((1, 8, 128), lambda b: (b, 0, 0)),
        out_shape=jax.ShapeDtypeStruct((B, 8, 128), jnp.float32),
    )(gmax)


def _key_of(v):
    bb = lax.bitcast_convert_type(v, jnp.int32)
    return bb ^ (lax.shift_right_arithmetic(bb, 31) & M31)


def _sc_body(g_hbm, th_hbm, s_hbm, x_hbm,
             vals_hbm, rows_hbm, cols_hbm, xi_hbm, xj_hbm,
             g_v, th_v, gid_v, elem_v, cval_v, cidx_v, c16_v,
             mv_v, mi_v, cnts_v, oval_v, orow_v, ocol_v,
             rl_v, xg_v,
             sp_val, sp_idx, sp_cnt, sp_sel,
             sem, sem2):
    c = lax.axis_index("c")
    s = lax.axis_index("s")
    lane = lax.iota(jnp.int32, 16)

    for q in range(4):
        b = c * 4 + q
        pltpu.sync_copy(th_hbm.at[b], th_v)
        th = th_v[pl.ds(0, 16)]
        pltpu.sync_copy(g_hbm.at[b, s], g_v)
        ninf = lax.broadcast(np.float32(-np.inf), (16,))
        zero16 = lax.broadcast(np.int32(0), (16,))

        def clr(i, _):
            cval_v[pl.ds(i * 16, 16)] = ninf
            cidx_v[pl.ds(i * 16, 16)] = zero16
            return np.int32(0)
        lax.fori_loop(0, CCAP // 16, clr, np.int32(0))

        gbase = s * GSH

        def gstep(i, ng):
            v = g_v[pl.ds(i * 16, 16)]
            m = v >= th
            cnt = jnp.sum((m).astype(jnp.int32))

            def take(ng):
                pos = ng + plsc.cumsum(m.astype(jnp.int32)) - 1
                gids = gbase + i * 16 + lane
                plsc.store_scatter(gid_v, [pos], gids, mask=m & (pos < GCAP))
                return jnp.minimum(ng + cnt, np.int32(GCAP))
            return lax.cond(cnt > 0, take, lambda ng: ng, ng)
        n_g = lax.fori_loop(0, GSH // 16, gstep, np.int32(0))

        sbase = b * LL

        def egather(i, _):
            gid = gid_v[pl.ds(i, 16)][0]
            r = lax.shift_right_logical(gid, 7)
            j = gid & 127
            eidx = sbase + r * L + j + lane * 128
            pltpu.async_copy(s_hbm.at[eidx], elem_v.at[pl.ds(i * 16, 16)],
                             sem).wait()
            return np.int32(0)
        lax.fori_loop(0, n_g, egather, np.int32(0))

        def estep(i, nc):
            v = elem_v[pl.ds(i * 16, 16)]
            gid = gid_v[pl.ds(i, 16)][0]
            r = lax.shift_right_logical(gid, 7)
            j = gid & 127
            fidx = r * L + j + lane * 128
            m = v >= th
            cnt = jnp.sum((m).astype(jnp.int32))
            pos = nc + plsc.cumsum(m.astype(jnp.int32)) - 1
            mm = m & (pos < CCAP)
            plsc.store_scatter(cval_v, [pos], v, mask=mm)
            plsc.store_scatter(cidx_v, [pos], fidx, mask=mm)
            return jnp.minimum(nc + cnt, np.int32(CCAP))
        n_c = lax.fori_loop(0, n_g, estep, np.int32(0))

        c16_v[...] = lax.broadcast(n_c, (16,))
        pltpu.sync_copy(cval_v, sp_val.at[q, s])
        pltpu.sync_copy(cidx_v, sp_idx.at[q, s])
        pltpu.sync_copy(c16_v, sp_cnt.at[q, pl.ds(s * 16, 16)])

    plsc.subcore_barrier()

    @pl.when(s < 4)
    def _():
        q = s
        b = c * 4 + q
        pltpu.sync_copy(sp_cnt.at[q], cnts_v)
        for w in range(16):
            pltpu.sync_copy(sp_val.at[q, w], mv_v.at[pl.ds(w * CCAP, CCAP)])
            pltpu.sync_copy(sp_idx.at[q, w], mi_v.at[pl.ds(w * CCAP, CCAP)])

        def fold(fn, init):
            acc = init
            for w in range(16):
                cw = cnts_v[pl.ds(w * 16, 16)][0]
                nv = lax.shift_right_logical(cw + 15, 4)

                def body(i, a, w=w):
                    off = w * CCAP + i * 16
                    return fn(a, mv_v[pl.ds(off, 16)], mi_v[pl.ds(off, 16)])
                acc = lax.fori_loop(0, nv, body, acc)
            return acc

        def bit_step(t, toff):
            bit = lax.shift_left(jnp.int32(1), 31 - t)
            cand = toff | bit
            thr = cand ^ I32_MIN
            cnt = fold(lambda a, v, ix: a + jnp.sum((_key_of(v) >= thr).astype(jnp.int32)),
                np.int32(0))
            return lax.select(cnt >= TOP_K, cand, toff)
        toff = lax.fori_loop(0, 32, bit_step, np.int32(0))
        vk = toff ^ I32_MIN

        c1 = fold(lambda a, v, ix: a + jnp.sum((_key_of(v) > vk).astype(jnp.int32)), np.int32(0))
        tneed = TOP_K - c1

        def ibit(t, res):
            bit = lax.shift_left(np.int32(1), 30 - t)
            cand = res | bit
            cl = fold(lambda a, v, ix: a + jnp.max(
                plsc.all_reduce_population_count(
                    (_key_of(v) == vk) & (ix < cand))), np.int32(0))
            return lax.select(cl < tneed, cand, res)
        res = lax.fori_loop(0, 31, ibit, np.int32(0))

        def emit(a, v, ix):
            k = _key_of(v)
            selm = (k > vk) | ((k == vk) & (ix <= res))
            pos = a + plsc.cumsum(selm.astype(jnp.int32)) - 1
            plsc.store_scatter(oval_v, [pos], v, mask=selm)
            plsc.store_scatter(orow_v, [pos],
                               lax.shift_right_logical(ix, 11), mask=selm)
            plsc.store_scatter(ocol_v, [pos], ix & jnp.int32(L - 1), mask=selm)
            return a + jnp.sum((selm).astype(jnp.int32))
        fold(emit, np.int32(0))

        pltpu.sync_copy(oval_v, vals_hbm.at[b])
        pltpu.sync_copy(orow_v, rows_hbm.at[b])
        pltpu.sync_copy(ocol_v, cols_hbm.at[b])
        pltpu.sync_copy(orow_v, sp_sel.at[q, 0])
        pltpu.sync_copy(ocol_v, sp_sel.at[q, 1])

    plsc.subcore_barrier()

    q2 = lax.div(s, np.int32(4))
    quar = lax.rem(s, np.int32(4))
    b2 = c * 4 + q2
    xbase = b2 * L
    for side in range(2):
        dst = xi_hbm if side == 0 else xj_hbm
        pltpu.sync_copy(sp_sel.at[q2, side, pl.ds(quar * 128, 128)], rl_v)
        for i in range(8):
            ridx = rl_v[pl.ds(i * 16, 16)] + xbase
            pltpu.async_copy(x_hbm.at[ridx], xg_v.at[pl.ds(i * 16, 16)],
                             sem2).wait()
        pltpu.sync_copy(xg_v, dst.at[b2, pl.ds(quar * 128, 128)])


def _sc_select(g2, th2, s_flat, x2):
    mesh = plsc.VectorSubcoreMesh(core_axis_name="c", subcore_axis_name="s",
                                  num_cores=2, num_subcores=16)
    return pl.kernel(
        _sc_body,
        out_type=(
            jax.ShapeDtypeStruct((B, TOP_K), jnp.float32),
            jax.ShapeDtypeStruct((B, TOP_K), jnp.int32),
            jax.ShapeDtypeStruct((B, TOP_K), jnp.int32),
            jax.ShapeDtypeStruct((B, TOP_K, D), jnp.float32),
            jax.ShapeDtypeStruct((B, TOP_K, D), jnp.float32),
        ),
        mesh=mesh,
        scratch_types=[
            pltpu.VMEM((GSH,), jnp.float32),
            pltpu.VMEM((1024,), jnp.float32),
            pltpu.VMEM((GCAP + 16,), jnp.int32),
            pltpu.VMEM((GCAP * 16,), jnp.float32),
            pltpu.VMEM((CCAP,), jnp.float32),
            pltpu.VMEM((CCAP,), jnp.int32),
            pltpu.VMEM((16,), jnp.int32),
            pltpu.VMEM((16 * CCAP,), jnp.float32),
            pltpu.VMEM((16 * CCAP,), jnp.int32),
            pltpu.VMEM((256,), jnp.int32),
            pltpu.VMEM((TOP_K,), jnp.float32),
            pltpu.VMEM((TOP_K,), jnp.int32),
            pltpu.VMEM((TOP_K,), jnp.int32),
            pltpu.VMEM((128,), jnp.int32),
            pltpu.VMEM((128, 128), jnp.float32),
            pltpu.VMEM_SHARED((4, 16, CCAP), jnp.float32),
            pltpu.VMEM_SHARED((4, 16, CCAP), jnp.int32),
            pltpu.VMEM_SHARED((4, 256), jnp.int32),
            pltpu.VMEM_SHARED((4, 2, TOP_K), jnp.int32),
            pltpu.SemaphoreType.DMA,
            pltpu.SemaphoreType.DMA,
        ],
        compiler_params=pltpu.CompilerParams(needs_layout_passes=False),
    )(g2, th2, s_flat, x2)


def _tail_body(vals_ref, rows_ref, cols_ref, xi_ref, xj_ref,
               phi1W_ref, phi1b_ref, phi2W_ref, phi2b_ref,
               xi1W_ref, xi1b_ref, xi2W_ref, xi2b_ref,
               rho1W_ref, rho1b_ref, rho2W_ref, rho2b_ref, o_ref):
    def dotg(a, w):
        return lax.dot_general(a, w, (((1,), (1,)), ((), ())),
                               preferred_element_type=jnp.float32)

    vals = vals_ref[...]
    m = jnp.max(vals, axis=-1, keepdims=True)
    e = jnp.exp(vals - m)
    w = e / jnp.sum(e, axis=-1, keepdims=True)

    xi = xi_ref[...].reshape(B * TOP_K, D)
    xj = xj_ref[...].reshape(B * TOP_K, D)
    xi1W = xi1W_ref[...]
    h1p = jnp.maximum(dotg(xi, xi1W[:, :D]) + dotg(xj, xi1W[:, D:])
                      + xi1b_ref[...], 0.0)
    fp = dotg(h1p, xi2W_ref[...]) + xi2b_ref[...]
    h1s = jnp.maximum(dotg(xi, phi1W_ref[...]) + phi1b_ref[...], 0.0)
    fs = dotg(h1s, phi2W_ref[...]) + phi2b_ref[...]
    sm = jnp.where(rows_ref[...] == cols_ref[...], 1.0, 0.0)
    ws = (w * sm)[:, :, None]
    wp = (w * (1.0 - sm))[:, :, None]
    pooled = jnp.sum(ws * fs.reshape(B, TOP_K, D)
                     + wp * fp.reshape(B, TOP_K, D), axis=1)
    h = jnp.maximum(dotg(pooled, rho1W_ref[...]) + rho1b_ref[...], 0.0)
    o_ref[...] = dotg(h, rho2W_ref[...]) + rho2b_ref[...]


def _tail(vals, rows, cols, xi, xj, phi1W, phi1b, phi2W, phi2b,
          xi1W, xi1b, xi2W, xi2b, rho1W, rho1b, rho2W, rho2b):
    return pl.pallas_call(
        _tail_body,
        out_shape=jax.ShapeDtypeStruct((B, H), jnp.float32),
    )(vals, rows, cols, xi, xj,
      phi1W, phi1b.reshape(1, H), phi2W, phi2b.reshape(1, H),
      xi1W, xi1b.reshape(1, H), xi2W, xi2b.reshape(1, H),
      rho1W, rho1b.reshape(1, H), rho2W, rho2b.reshape(1, H))


def kernel(x, qW, qb, kW, kb, phi1W, phi1b, phi2W, phi2b,
           xi1W, xi1b, xi2W, xi2b, rho1W, rho1b, rho2W, rho2b):
    scores, gmax = _scores(x, qW, qb, kW, kb)
    theta = _theta(gmax)
    vals, rows, cols, xi, xj = _sc_select(
        gmax.reshape(B, 16, GSH), theta.reshape(B, 1024),
        scores.reshape(B * LL), x.reshape(B * L, D))
    return _tail(vals, rows, cols, xi, xj, phi1W, phi1b, phi2W, phi2b,
                 xi1W, xi1b, xi2W, xi2b, rho1W, rho1b, rho2W, rho2b)


# speedup vs baseline: 98.4011x; 1.2857x over previous
import numpy as np

import jax
import jax.numpy as jnp
from jax import lax
from jax.experimental import pallas as pl
from jax.experimental.pallas import tpu as pltpu
from jax.experimental.pallas import tpu_sc as plsc

B, L, D = 8, 2048, 128
H = 128
TOP_K = 512
SCALE = H ** -0.5
LL = L * L
NG = L * 128
GSH = NG // 16
GCAP = 512
CCAP = 1024
I32_MIN = -2147483648
M31 = 0x7FFFFFFF


def _scores_body(x_ref, qW_ref, qb_ref, kW_ref, kb_ref, s_ref, g_ref):
    xb = x_ref[0]
    q = lax.dot_general(xb, qW_ref[...], (((1,), (1,)), ((), ())),
                        preferred_element_type=jnp.float32) + qb_ref[...]
    k = lax.dot_general(xb, kW_ref[...], (((1,), (1,)), ((), ())),
                        preferred_element_type=jnp.float32) + kb_ref[...]
    valid = jnp.sum(jnp.abs(xb), axis=1) != 0.0
    g = None
    for l in range(16):
        kl = k[128 * l:128 * (l + 1), :]
        sl = lax.dot_general(q, kl, (((1,), (1,)), ((), ())),
                             preferred_element_type=jnp.float32) * SCALE
        m2 = valid[:, None] & valid[None, 128 * l:128 * (l + 1)]
        sl = jnp.where(m2, sl, -jnp.inf)
        s_ref[0, pl.ds(l * L, L)] = sl
        g = sl if g is None else jnp.maximum(g, sl)
    g_ref[0] = g


def _scores(x, qW, qb, kW, kb):
    return pl.pallas_call(
        _scores_body,
        grid=(B,),
        in_specs=[
            pl.BlockSpec((1, L, D), lambda b: (b, 0, 0)),
            pl.BlockSpec((H, D), lambda b: (0, 0)),
            pl.BlockSpec((1, H), lambda b: (0, 0)),
            pl.BlockSpec((H, D), lambda b: (0, 0)),
            pl.BlockSpec((1, H), lambda b: (0, 0)),
        ],
        out_specs=[
            pl.BlockSpec((1, 16 * L, 128), lambda b: (b, 0, 0)),
            pl.BlockSpec((1, L, 128), lambda b: (b, 0, 0)),
        ],
        out_shape=[
            jax.ShapeDtypeStruct((B, 16 * L, 128), jnp.float32),
            jax.ShapeDtypeStruct((B, L, 128), jnp.float32),
        ],
    )(x, qW, qb.reshape(1, H), kW, kb.reshape(1, H))


def _theta_body(g_ref, th_ref):
    g = g_ref[0]
    bb = lax.bitcast_convert_type(g, jnp.int32)
    keys = bb ^ (lax.shift_right_arithmetic(bb, 31) & M31)

    def bit_step(i, toff):
        bit = lax.shift_left(jnp.int32(1), 31 - i)
        cand = toff | bit
        thr = cand ^ I32_MIN
        cnt = jnp.sum((keys >= thr).astype(jnp.int32))
        return lax.select(cnt >= TOP_K, cand, toff)

    toff = lax.fori_loop(0, 32, bit_step, np.int32(0))
    vk = toff ^ I32_MIN
    fb = jnp.where(vk < 0, vk ^ M31, vk)
    th = lax.bitcast_convert_type(fb, jnp.float32)
    th_ref[0] = jnp.full((8, 128), th, jnp.float32)


def _theta(gmax):
    return pl.pallas_call(
        _theta_body,
        grid=(B,),
        in_specs=[pl.BlockSpec((1, L, 128), lambda b: (b, 0, 0))],
        out_specs=pl.BlockSpec((1, 8, 128), lambda b: (b, 0, 0)),
        out_shape=jax.ShapeDtypeStruct((B, 8, 128), jnp.float32),
    )(gmax)


def _key_of(v):
    bb = lax.bitcast_convert_type(v, jnp.int32)
    return bb ^ (lax.shift_right_arithmetic(bb, 31) & M31)


def _sc_body(g_hbm, th_hbm, s_hbm, x_hbm,
             vals_hbm, rows_hbm, cols_hbm, xi_hbm, xj_hbm,
             g_v, th_v, gid_v, elem_v, cval_v, cidx_v, c16_v,
             mv_v, mi_v, cnts_v, oval_v, orow_v, ocol_v,
             rl_v, xg_v,
             sp_val, sp_idx, sp_cnt, sp_sel,
             sem, sem2):
    c = lax.axis_index("c")
    s = lax.axis_index("s")
    lane = lax.iota(jnp.int32, 16)

    for q in range(4):
        b = c * 4 + q
        pltpu.sync_copy(th_hbm.at[pl.ds(b * 1024, 16)], th_v)
        th = th_v[pl.ds(0, 16)]
        pltpu.sync_copy(g_hbm.at[pl.ds(b * NG + s * GSH, GSH)], g_v)
        ninf = lax.broadcast(np.float32(-np.inf), (16,))
        zero16 = lax.broadcast(np.int32(0), (16,))

        def clr(i, _):
            for u in range(8):
                cval_v[pl.ds((i * 8 + u) * 16, 16)] = ninf
                cidx_v[pl.ds((i * 8 + u) * 16, 16)] = zero16
            return np.int32(0)
        lax.fori_loop(0, CCAP // 128, clr, np.int32(0))

        gbase = s * GSH

        def gstep(i, ng):
            ms = []
            for u in range(8):
                v = g_v[pl.ds((i * 8 + u) * 16, 16)]
                ms.append(v >= th)
            anym = ms[0]
            for u in range(1, 8):
                anym = anym | ms[u]
            any_cnt = jnp.sum(jnp.where(anym, 1, 0))

            def take(ng):
                for u in range(8):
                    m = ms[u]
                    cnt = jnp.sum(jnp.where(m, 1, 0))
                    pos = ng + plsc.cumsum(jnp.where(m, 1, 0)) - 1
                    gids = gbase + (i * 8 + u) * 16 + lane
                    plsc.store_scatter(gid_v, [pos], gids,
                                       mask=m & (pos < GCAP))
                    ng = jnp.minimum(ng + cnt, np.int32(GCAP))
                return ng
            return lax.cond(any_cnt > 0, take, lambda ng: ng, ng)
        n_g = lax.fori_loop(0, GSH // 128, gstep, np.int32(0))

        sbase = b * LL

        def egather(i, _):
            gid = gid_v[pl.ds(i, 16)][0]
            r = lax.shift_right_logical(gid, 7)
            j = gid & 127
            eidx = sbase + lane * (L * 128) + r * 128 + j
            pltpu.async_copy(s_hbm.at[eidx], elem_v.at[pl.ds(i * 16, 16)],
                             sem).wait()
            return np.int32(0)
        lax.fori_loop(0, n_g, egather, np.int32(0))

        def estep(i, nc):
            v = elem_v[pl.ds(i * 16, 16)]
            gid = gid_v[pl.ds(i, 16)][0]
            r = lax.shift_right_logical(gid, 7)
            j = gid & 127
            fidx = r * L + j + lane * 128
            m = v >= th
            cnt = jnp.sum((m).astype(jnp.int32))
            pos = nc + plsc.cumsum(m.astype(jnp.int32)) - 1
            mm = m & (pos < CCAP)
            plsc.store_scatter(cval_v, [pos], v, mask=mm)
            plsc.store_scatter(cidx_v, [pos], fidx, mask=mm)
            return jnp.minimum(nc + cnt, np.int32(CCAP))
        n_c = lax.fori_loop(0, n_g, estep, np.int32(0))

        c16_v[...] = lax.broadcast(n_c, (16,))
        pltpu.sync_copy(cval_v, sp_val.at[q, s])
        pltpu.sync_copy(cidx_v, sp_idx.at[q, s])
        pltpu.sync_copy(c16_v, sp_cnt.at[q, pl.ds(s * 16, 16)])

    plsc.subcore_barrier()

    @pl.when(s < 4)
    def _():
        q = s
        b = c * 4 + q
        pltpu.sync_copy(sp_cnt.at[q], cnts_v)
        for w in range(16):
            pltpu.sync_copy(sp_val.at[q, w], mv_v.at[pl.ds(w * CCAP, CCAP)])
            pltpu.sync_copy(sp_idx.at[q, w], mi_v.at[pl.ds(w * CCAP, CCAP)])

        def fold(fn, init):
            acc = init
            for w in range(16):
                cw = cnts_v[pl.ds(w * 16, 16)][0]
                nv = lax.shift_right_logical(cw + 15, 4)

                def body(i, a, w=w):
                    off = w * CCAP + i * 16
                    return fn(a, mv_v[pl.ds(off, 16)], mi_v[pl.ds(off, 16)])
                acc = lax.fori_loop(0, nv, body, acc)
            return acc

        def bit_step(t, toff):
            bit = lax.shift_left(jnp.int32(1), 31 - t)
            cand = toff | bit
            thr = cand ^ I32_MIN
            cnt = fold(lambda a, v, ix: a + jnp.sum((_key_of(v) >= thr).astype(jnp.int32)),
                np.int32(0))
            return lax.select(cnt >= TOP_K, cand, toff)
        toff = lax.fori_loop(0, 32, bit_step, np.int32(0))
        vk = toff ^ I32_MIN

        c1 = fold(lambda a, v, ix: a + jnp.sum((_key_of(v) > vk).astype(jnp.int32)), np.int32(0))
        tneed = TOP_K - c1

        def ibit(t, res):
            bit = lax.shift_left(np.int32(1), 30 - t)
            cand = res | bit
            cl = fold(lambda a, v, ix: a + jnp.max(
                plsc.all_reduce_population_count(
                    (_key_of(v) == vk) & (ix < cand))), np.int32(0))
            return lax.select(cl < tneed, cand, res)
        res = lax.fori_loop(0, 31, ibit, np.int32(0))

        def emit(a, v, ix):
            k = _key_of(v)
            selm = (k > vk) | ((k == vk) & (ix <= res))
            pos = a + plsc.cumsum(selm.astype(jnp.int32)) - 1
            plsc.store_scatter(oval_v, [pos], v, mask=selm)
            plsc.store_scatter(orow_v, [pos],
                               lax.shift_right_logical(ix, 11), mask=selm)
            plsc.store_scatter(ocol_v, [pos], ix & jnp.int32(L - 1), mask=selm)
            return a + jnp.sum((selm).astype(jnp.int32))
        fold(emit, np.int32(0))

        pltpu.sync_copy(oval_v, vals_hbm.at[b])
        pltpu.sync_copy(orow_v, rows_hbm.at[b])
        pltpu.sync_copy(ocol_v, cols_hbm.at[b])
        pltpu.sync_copy(orow_v, sp_sel.at[q, 0])
        pltpu.sync_copy(ocol_v, sp_sel.at[q, 1])

    plsc.subcore_barrier()

    q2 = lax.div(s, np.int32(4))
    quar = lax.rem(s, np.int32(4))
    b2 = c * 4 + q2
    xbase = b2 * L
    for side in range(2):
        dst = xi_hbm if side == 0 else xj_hbm
        pltpu.sync_copy(sp_sel.at[q2, side, pl.ds(quar * 128, 128)], rl_v)
        for i in range(8):
            ridx = rl_v[pl.ds(i * 16, 16)] + xbase
            pltpu.async_copy(x_hbm.at[ridx], xg_v.at[pl.ds(i * 16, 16)],
                             sem2).wait()
        pltpu.sync_copy(xg_v, dst.at[b2, pl.ds(quar * 128, 128)])


def _sc_select(g2, th2, s_flat, x2):
    mesh = plsc.VectorSubcoreMesh(core_axis_name="c", subcore_axis_name="s",
                                  num_cores=2, num_subcores=16)
    return pl.kernel(
        _sc_body,
        out_type=(
            jax.ShapeDtypeStruct((B, TOP_K), jnp.float32),
            jax.ShapeDtypeStruct((B, TOP_K), jnp.int32),
            jax.ShapeDtypeStruct((B, TOP_K), jnp.int32),
            jax.ShapeDtypeStruct((B, TOP_K, D), jnp.float32),
            jax.ShapeDtypeStruct((B, TOP_K, D), jnp.float32),
        ),
        mesh=mesh,
        scratch_types=[
            pltpu.VMEM((GSH,), jnp.float32),
            pltpu.VMEM((16,), jnp.float32),
            pltpu.VMEM((GCAP + 16,), jnp.int32),
            pltpu.VMEM((GCAP * 16,), jnp.float32),
            pltpu.VMEM((CCAP,), jnp.float32),
            pltpu.VMEM((CCAP,), jnp.int32),
            pltpu.VMEM((16,), jnp.int32),
            pltpu.VMEM((16 * CCAP,), jnp.float32),
            pltpu.VMEM((16 * CCAP,), jnp.int32),
            pltpu.VMEM((256,), jnp.int32),
            pltpu.VMEM((TOP_K,), jnp.float32),
            pltpu.VMEM((TOP_K,), jnp.int32),
            pltpu.VMEM((TOP_K,), jnp.int32),
            pltpu.VMEM((128,), jnp.int32),
            pltpu.VMEM((128, 128), jnp.float32),
            pltpu.VMEM_SHARED((4, 16, CCAP), jnp.float32),
            pltpu.VMEM_SHARED((4, 16, CCAP), jnp.int32),
            pltpu.VMEM_SHARED((4, 256), jnp.int32),
            pltpu.VMEM_SHARED((4, 2, TOP_K), jnp.int32),
            pltpu.SemaphoreType.DMA,
            pltpu.SemaphoreType.DMA,
        ],
        compiler_params=pltpu.CompilerParams(needs_layout_passes=False),
    )(g2, th2, s_flat, x2)


def _tail_body(vals_ref, rows_ref, cols_ref, xi_ref, xj_ref,
               phi1W_ref, phi1b_ref, phi2W_ref, phi2b_ref,
               xi1W_ref, xi1b_ref, xi2W_ref, xi2b_ref,
               rho1W_ref, rho1b_ref, rho2W_ref, rho2b_ref, o_ref):
    def dotg(a, w):
        return lax.dot_general(a, w, (((1,), (1,)), ((), ())),
                               preferred_element_type=jnp.float32)

    vals = vals_ref[...]
    m = jnp.max(vals, axis=-1, keepdims=True)
    e = jnp.exp(vals - m)
    w = e / jnp.sum(e, axis=-1, keepdims=True)

    xi = xi_ref[...].reshape(B * TOP_K, D)
    xj = xj_ref[...].reshape(B * TOP_K, D)
    xi1W = xi1W_ref[...]
    h1p = jnp.maximum(dotg(xi, xi1W[:, :D]) + dotg(xj, xi1W[:, D:])
                      + xi1b_ref[...], 0.0)
    fp = dotg(h1p, xi2W_ref[...]) + xi2b_ref[...]
    h1s = jnp.maximum(dotg(xi, phi1W_ref[...]) + phi1b_ref[...], 0.0)
    fs = dotg(h1s, phi2W_ref[...]) + phi2b_ref[...]
    sm = jnp.where(rows_ref[...] == cols_ref[...], 1.0, 0.0)
    ws = (w * sm)[:, :, None]
    wp = (w * (1.0 - sm))[:, :, None]
    pooled = jnp.sum(ws * fs.reshape(B, TOP_K, D)
                     + wp * fp.reshape(B, TOP_K, D), axis=1)
    h = jnp.maximum(dotg(pooled, rho1W_ref[...]) + rho1b_ref[...], 0.0)
    o_ref[...] = dotg(h, rho2W_ref[...]) + rho2b_ref[...]


def _tail(vals, rows, cols, xi, xj, phi1W, phi1b, phi2W, phi2b,
          xi1W, xi1b, xi2W, xi2b, rho1W, rho1b, rho2W, rho2b):
    return pl.pallas_call(
        _tail_body,
        out_shape=jax.ShapeDtypeStruct((B, H), jnp.float32),
    )(vals, rows, cols, xi, xj,
      phi1W, phi1b.reshape(1, H), phi2W, phi2b.reshape(1, H),
      xi1W, xi1b.reshape(1, H), xi2W, xi2b.reshape(1, H),
      rho1W, rho1b.reshape(1, H), rho2W, rho2b.reshape(1, H))


def kernel(x, qW, qb, kW, kb, phi1W, phi1b, phi2W, phi2b,
           xi1W, xi1b, xi2W, xi2b, rho1W, rho1b, rho2W, rho2b):
    scores, gmax = _scores(x, qW, qb, kW, kb)
    theta = _theta(gmax)
    vals, rows, cols, xi, xj = _sc_select(
        gmax.reshape(B * NG), theta.reshape(B * 1024),
        scores.reshape(B * LL), x.reshape(B * L, D))
    return _tail(vals, rows, cols, xi, xj, phi1W, phi1b, phi2W, phi2b,
                 xi1W, xi1b, xi2W, xi2b, rho1W, rho1b, rho2W, rho2b)


# speedup vs baseline: 151.6908x; 1.5416x over previous
import numpy as np

import jax
import jax.numpy as jnp
from jax import lax
from jax.experimental import pallas as pl
from jax.experimental.pallas import tpu as pltpu
from jax.experimental.pallas import tpu_sc as plsc

B, L, D = 8, 2048, 128
H = 128
TOP_K = 512
SCALE = H ** -0.5
LL = L * L
NG = L * 128
GSH = NG // 16
GCAP = 512
CCAP = 1024
I32_MIN = -2147483648
M31 = 0x7FFFFFFF


def _scores_body(x_ref, qW_ref, qb_ref, kW_ref, kb_ref, s_ref, g_ref, th_ref):
    xb = x_ref[0]
    q = lax.dot_general(xb, qW_ref[...], (((1,), (1,)), ((), ())),
                        preferred_element_type=jnp.float32) + qb_ref[...]
    k = lax.dot_general(xb, kW_ref[...], (((1,), (1,)), ((), ())),
                        preferred_element_type=jnp.float32) + kb_ref[...]
    valid = jnp.sum(jnp.abs(xb), axis=1) != 0.0
    g = None
    for l in range(16):
        kl = k[128 * l:128 * (l + 1), :]
        sl = lax.dot_general(q, kl, (((1,), (1,)), ((), ())),
                             preferred_element_type=jnp.float32) * SCALE
        m2 = valid[:, None] & valid[None, 128 * l:128 * (l + 1)]
        sl = jnp.where(m2, sl, -jnp.inf)
        s_ref[0, pl.ds(l * L, L)] = sl
        g = sl if g is None else jnp.maximum(g, sl)
    g_ref[0] = g
    m8 = jnp.max(g.reshape(L // 8, 8, 128), axis=1)
    bb = lax.bitcast_convert_type(m8, jnp.int32)
    keys = bb ^ (lax.shift_right_arithmetic(bb, 31) & M31)

    def bit_step(i, toff):
        bit = lax.shift_left(jnp.int32(1), 31 - i)
        cand = toff | bit
        thr = cand ^ I32_MIN
        cnt = jnp.sum((keys >= thr).astype(jnp.int32))
        return lax.select(cnt >= TOP_K, cand, toff)

    toff = lax.fori_loop(0, 32, bit_step, np.int32(0))
    vk = toff ^ I32_MIN
    fb = jnp.where(vk < 0, vk ^ M31, vk)
    th = lax.bitcast_convert_type(fb, jnp.float32)
    th_ref[0] = jnp.full((8, 128), th, jnp.float32)


def _scores(x, qW, qb, kW, kb):
    return pl.pallas_call(
        _scores_body,
        grid=(B,),
        in_specs=[
            pl.BlockSpec((1, L, D), lambda b: (b, 0, 0)),
            pl.BlockSpec((H, D), lambda b: (0, 0)),
            pl.BlockSpec((1, H), lambda b: (0, 0)),
            pl.BlockSpec((H, D), lambda b: (0, 0)),
            pl.BlockSpec((1, H), lambda b: (0, 0)),
        ],
        out_specs=[
            pl.BlockSpec((1, 16 * L, 128), lambda b: (b, 0, 0)),
            pl.BlockSpec((1, L, 128), lambda b: (b, 0, 0)),
            pl.BlockSpec((1, 8, 128), lambda b: (b, 0, 0)),
        ],
        out_shape=[
            jax.ShapeDtypeStruct((B, 16 * L, 128), jnp.float32),
            jax.ShapeDtypeStruct((B, L, 128), jnp.float32),
            jax.ShapeDtypeStruct((B, 8, 128), jnp.float32),
        ],
    )(x, qW, qb.reshape(1, H), kW, kb.reshape(1, H))


def _key_of(v):
    bb = lax.bitcast_convert_type(v, jnp.int32)
    return bb ^ (lax.shift_right_arithmetic(bb, 31) & M31)


def _sc_body(g_hbm, th_hbm, s_hbm, x_hbm,
             vals_hbm, rows_hbm, cols_hbm, xi_hbm, xj_hbm,
             g_v, th_v, gid_v, elem_v, cval_v, cidx_v, c16_v,
             mv_v, mi_v, cnts_v, oval_v, orow_v, ocol_v,
             rl_v, xg_v,
             sp_val, sp_idx, sp_cnt, sp_sel,
             sem, sem2):
    c = lax.axis_index("c")
    s = lax.axis_index("s")
    lane = lax.iota(jnp.int32, 16)

    for q in range(4):
        b = c * 4 + q
        pltpu.sync_copy(th_hbm.at[pl.ds(b * 1024, 16)], th_v)
        th = th_v[pl.ds(0, 16)]
        pltpu.sync_copy(g_hbm.at[pl.ds(b * NG + s * GSH, GSH)], g_v)
        ninf = lax.broadcast(np.float32(-np.inf), (16,))
        zero16 = lax.broadcast(np.int32(0), (16,))

        def clr(i, _):
            for u in range(8):
                cval_v[pl.ds((i * 8 + u) * 16, 16)] = ninf
                cidx_v[pl.ds((i * 8 + u) * 16, 16)] = zero16
            return np.int32(0)
        lax.fori_loop(0, CCAP // 128, clr, np.int32(0))

        gbase = s * GSH

        def gstep(i, ng):
            ms = []
            for u in range(8):
                v = g_v[pl.ds((i * 8 + u) * 16, 16)]
                ms.append(v >= th)
            anym = ms[0]
            for u in range(1, 8):
                anym = anym | ms[u]
            any_cnt = jnp.sum(jnp.where(anym, 1, 0))

            def take(ng):
                for u in range(8):
                    m = ms[u]
                    cnt = jnp.sum(jnp.where(m, 1, 0))
                    pos = ng + plsc.cumsum(jnp.where(m, 1, 0)) - 1
                    gids = gbase + (i * 8 + u) * 16 + lane
                    plsc.store_scatter(gid_v, [pos], gids,
                                       mask=m & (pos < GCAP))
                    ng = jnp.minimum(ng + cnt, np.int32(GCAP))
                return ng
            return lax.cond(any_cnt > 0, take, lambda ng: ng, ng)
        n_g = lax.fori_loop(0, GSH // 128, gstep, np.int32(0))

        sbase = b * LL

        def efire(i, _):
            gid = gid_v[pl.ds(i, 16)][0]
            r = lax.shift_right_logical(gid, 7)
            j = gid & 127
            eidx = sbase + lane * (L * 128) + r * 128 + j
            pltpu.async_copy(s_hbm.at[eidx], elem_v.at[pl.ds(i * 16, 16)], sem)
            return np.int32(0)
        lax.fori_loop(0, n_g, efire, np.int32(0))

        def edrain(i, _):
            pltpu.make_async_copy(s_hbm.at[pl.ds(0, 16)],
                                  elem_v.at[pl.ds(i * 16, 16)], sem).wait()
            return np.int32(0)
        lax.fori_loop(0, n_g, edrain, np.int32(0))

        def estep(i, nc):
            v = elem_v[pl.ds(i * 16, 16)]
            gid = gid_v[pl.ds(i, 16)][0]
            r = lax.shift_right_logical(gid, 7)
            j = gid & 127
            fidx = r * L + j + lane * 128
            m = v >= th
            cnt = jnp.sum((m).astype(jnp.int32))
            pos = nc + plsc.cumsum(m.astype(jnp.int32)) - 1
            mm = m & (pos < CCAP)
            plsc.store_scatter(cval_v, [pos], v, mask=mm)
            plsc.store_scatter(cidx_v, [pos], fidx, mask=mm)
            return jnp.minimum(nc + cnt, np.int32(CCAP))
        n_c = lax.fori_loop(0, n_g, estep, np.int32(0))

        c16_v[...] = lax.broadcast(n_c, (16,))
        pltpu.sync_copy(cval_v, sp_val.at[q, s])
        pltpu.sync_copy(cidx_v, sp_idx.at[q, s])
        pltpu.sync_copy(c16_v, sp_cnt.at[q, pl.ds(s * 16, 16)])

    plsc.subcore_barrier()

    @pl.when(s < 4)
    def _():
        q = s
        b = c * 4 + q
        pltpu.sync_copy(sp_cnt.at[q], cnts_v)
        for w in range(16):
            pltpu.sync_copy(sp_val.at[q, w], mv_v.at[pl.ds(w * CCAP, CCAP)])
            pltpu.sync_copy(sp_idx.at[q, w], mi_v.at[pl.ds(w * CCAP, CCAP)])

        def fold(fn, init):
            acc = init
            for w in range(16):
                cw = cnts_v[pl.ds(w * 16, 16)][0]
                nv = lax.shift_right_logical(cw + 15, 4)

                def body(i, a, w=w):
                    off = w * CCAP + i * 16
                    return fn(a, mv_v[pl.ds(off, 16)], mi_v[pl.ds(off, 16)])
                acc = lax.fori_loop(0, nv, body, acc)
            return acc

        zeros16 = lax.broadcast(np.int32(0), (16,))

        def bit_step(t, toff):
            bit = lax.shift_left(jnp.int32(1), 31 - t)
            cand = toff | bit
            thr = cand ^ I32_MIN
            accv = fold(lambda a, v, ix: a + jnp.where(_key_of(v) >= thr, 1, 0),
                        zeros16)
            return lax.select(jnp.sum(accv) >= TOP_K, cand, toff)
        toff = lax.fori_loop(0, 32, bit_step, np.int32(0))
        vk = toff ^ I32_MIN

        c1 = jnp.sum(fold(lambda a, v, ix: a + jnp.where(_key_of(v) > vk, 1, 0),
                          zeros16))
        tneed = TOP_K - c1

        def ibit(t, res):
            bit = lax.shift_left(np.int32(1), 21 - t)
            cand = res | bit
            accv = fold(lambda a, v, ix: a + jnp.where(
                (_key_of(v) == vk) & (ix < cand), 1, 0), zeros16)
            return lax.select(jnp.sum(accv) < tneed, cand, res)
        res = lax.fori_loop(0, 22, ibit, np.int32(0))

        def emit(a, v, ix):
            k = _key_of(v)
            selm = (k > vk) | ((k == vk) & (ix <= res))
            pos = a + plsc.cumsum(selm.astype(jnp.int32)) - 1
            plsc.store_scatter(oval_v, [pos], v, mask=selm)
            plsc.store_scatter(orow_v, [pos],
                               lax.shift_right_logical(ix, 11), mask=selm)
            plsc.store_scatter(ocol_v, [pos], ix & jnp.int32(L - 1), mask=selm)
            return a + jnp.sum((selm).astype(jnp.int32))
        fold(emit, np.int32(0))

        pltpu.sync_copy(oval_v, vals_hbm.at[b])
        pltpu.sync_copy(orow_v, rows_hbm.at[b])
        pltpu.sync_copy(ocol_v, cols_hbm.at[b])
        pltpu.sync_copy(orow_v, sp_sel.at[q, 0])
        pltpu.sync_copy(ocol_v, sp_sel.at[q, 1])

    plsc.subcore_barrier()

    q2 = lax.div(s, np.int32(4))
    quar = lax.rem(s, np.int32(4))
    b2 = c * 4 + q2
    xbase = b2 * L
    for side in range(2):
        dst = xi_hbm if side == 0 else xj_hbm
        pltpu.sync_copy(sp_sel.at[q2, side, pl.ds(quar * 128, 128)], rl_v)
        for i in range(8):
            ridx = rl_v[pl.ds(i * 16, 16)] + xbase
            pltpu.async_copy(x_hbm.at[ridx], xg_v.at[pl.ds(i * 16, 16)], sem2)
        for i in range(8):
            pltpu.make_async_copy(x_hbm.at[pl.ds(0, 16)],
                                  xg_v.at[pl.ds(i * 16, 16)], sem2).wait()
        pltpu.sync_copy(xg_v, dst.at[b2, pl.ds(quar * 128, 128)])


def _sc_select(g2, th2, s_flat, x2):
    mesh = plsc.VectorSubcoreMesh(core_axis_name="c", subcore_axis_name="s",
                                  num_cores=2, num_subcores=16)
    return pl.kernel(
        _sc_body,
        out_type=(
            jax.ShapeDtypeStruct((B, TOP_K), jnp.float32),
            jax.ShapeDtypeStruct((B, TOP_K), jnp.int32),
            jax.ShapeDtypeStruct((B, TOP_K), jnp.int32),
            jax.ShapeDtypeStruct((B, TOP_K, D), jnp.float32),
            jax.ShapeDtypeStruct((B, TOP_K, D), jnp.float32),
        ),
        mesh=mesh,
        scratch_types=[
            pltpu.VMEM((GSH,), jnp.float32),
            pltpu.VMEM((16,), jnp.float32),
            pltpu.VMEM((GCAP + 16,), jnp.int32),
            pltpu.VMEM((GCAP * 16,), jnp.float32),
            pltpu.VMEM((CCAP,), jnp.float32),
            pltpu.VMEM((CCAP,), jnp.int32),
            pltpu.VMEM((16,), jnp.int32),
            pltpu.VMEM((16 * CCAP,), jnp.float32),
            pltpu.VMEM((16 * CCAP,), jnp.int32),
            pltpu.VMEM((256,), jnp.int32),
            pltpu.VMEM((TOP_K,), jnp.float32),
            pltpu.VMEM((TOP_K,), jnp.int32),
            pltpu.VMEM((TOP_K,), jnp.int32),
            pltpu.VMEM((128,), jnp.int32),
            pltpu.VMEM((128, 128), jnp.float32),
            pltpu.VMEM_SHARED((4, 16, CCAP), jnp.float32),
            pltpu.VMEM_SHARED((4, 16, CCAP), jnp.int32),
            pltpu.VMEM_SHARED((4, 256), jnp.int32),
            pltpu.VMEM_SHARED((4, 2, TOP_K), jnp.int32),
            pltpu.SemaphoreType.DMA,
            pltpu.SemaphoreType.DMA,
        ],
        compiler_params=pltpu.CompilerParams(needs_layout_passes=False),
    )(g2, th2, s_flat, x2)


def _tail_body(vals_ref, rows_ref, cols_ref, xi_ref, xj_ref,
               phi1W_ref, phi1b_ref, phi2W_ref, phi2b_ref,
               xi1W_ref, xi1b_ref, xi2W_ref, xi2b_ref,
               rho1W_ref, rho1b_ref, rho2W_ref, rho2b_ref, o_ref):
    def dotg(a, w):
        return lax.dot_general(a, w, (((1,), (1,)), ((), ())),
                               preferred_element_type=jnp.float32)

    vals = vals_ref[...]
    m = jnp.max(vals, axis=-1, keepdims=True)
    e = jnp.exp(vals - m)
    w = e / jnp.sum(e, axis=-1, keepdims=True)

    xi = xi_ref[...].reshape(B * TOP_K, D)
    xj = xj_ref[...].reshape(B * TOP_K, D)
    xi1W = xi1W_ref[...]
    h1p = jnp.maximum(dotg(xi, xi1W[:, :D]) + dotg(xj, xi1W[:, D:])
                      + xi1b_ref[...], 0.0)
    fp = dotg(h1p, xi2W_ref[...]) + xi2b_ref[...]
    h1s = jnp.maximum(dotg(xi, phi1W_ref[...]) + phi1b_ref[...], 0.0)
    fs = dotg(h1s, phi2W_ref[...]) + phi2b_ref[...]
    sm = jnp.where(rows_ref[...] == cols_ref[...], 1.0, 0.0)
    ws = (w * sm)[:, :, None]
    wp = (w * (1.0 - sm))[:, :, None]
    pooled = jnp.sum(ws * fs.reshape(B, TOP_K, D)
                     + wp * fp.reshape(B, TOP_K, D), axis=1)
    h = jnp.maximum(dotg(pooled, rho1W_ref[...]) + rho1b_ref[...], 0.0)
    o_ref[...] = dotg(h, rho2W_ref[...]) + rho2b_ref[...]


def _tail(vals, rows, cols, xi, xj, phi1W, phi1b, phi2W, phi2b,
          xi1W, xi1b, xi2W, xi2b, rho1W, rho1b, rho2W, rho2b):
    return pl.pallas_call(
        _tail_body,
        out_shape=jax.ShapeDtypeStruct((B, H), jnp.float32),
    )(vals, rows, cols, xi, xj,
      phi1W, phi1b.reshape(1, H), phi2W, phi2b.reshape(1, H),
      xi1W, xi1b.reshape(1, H), xi2W, xi2b.reshape(1, H),
      rho1W, rho1b.reshape(1, H), rho2W, rho2b.reshape(1, H))


def kernel(x, qW, qb, kW, kb, phi1W, phi1b, phi2W, phi2b,
           xi1W, xi1b, xi2W, xi2b, rho1W, rho1b, rho2W, rho2b):
    scores, gmax, theta = _scores(x, qW, qb, kW, kb)
    vals, rows, cols, xi, xj = _sc_select(
        gmax.reshape(B * NG), theta.reshape(B * 1024),
        scores.reshape(B * LL), x.reshape(B * L, D))
    return _tail(vals, rows, cols, xi, xj, phi1W, phi1b, phi2W, phi2b,
                 xi1W, xi1b, xi2W, xi2b, rho1W, rho1b, rho2W, rho2b)
